# baseline (speedup 1.0000x reference)
import jax
import jax.numpy as jnp
from jax.experimental import pallas as pl
from jax.experimental.pallas import tpu as pltpu


def _pool_kernel(feat_ref, mask_ref, out_ref):
    feat = feat_ref[...].astype(jnp.float32)
    m2 = mask_ref[...].astype(jnp.float32)
    mask = m2[:, :, None]
    num = jnp.sum(feat * mask, axis=1)
    den = jnp.sum(m2, axis=1, keepdims=True)
    out_ref[...] = (num * (1.0 / den)).astype(out_ref.dtype)


def kernel(features, input_mask):
    B, S, H = features.shape
    itemsize = jnp.dtype(features.dtype).itemsize

    tb = B
    for cand in (8, 4, 2, 1):
        blk = cand * S * H * itemsize
        if B % cand == 0 and 2 * blk <= 49 << 20:
            tb = cand
            break

    grid = (B // tb,)
    feat_spec = pl.BlockSpec((tb, S, H), lambda i: (i, 0, 0))
    mask_spec = pl.BlockSpec((tb, S), lambda i: (i, 0))
    out_spec = pl.BlockSpec((tb, H), lambda i: (i, 0))

    feat_blk = tb * S * H * itemsize
    vmem = min(56 << 20, 2 * feat_blk + (8 << 20))

    return pl.pallas_call(
        _pool_kernel,
        out_shape=jax.ShapeDtypeStruct((B, H), features.dtype),
        grid=grid,
        in_specs=[feat_spec, mask_spec],
        out_specs=out_spec,
        compiler_params=pltpu.CompilerParams(
            dimension_semantics=("parallel",),
            vmem_limit_bytes=int(vmem),
        ),
        cost_estimate=pl.CostEstimate(
            flops=2 * B * S * H,
            transcendentals=0,
            bytes_accessed=B * S * H * itemsize + B * S * 4 + B * H * itemsize,
        ),
    )(features, input_mask)


# speedup vs baseline: 1.0036x; 1.0036x over previous
import jax
import jax.numpy as jnp
from jax.experimental import pallas as pl
from jax.experimental.pallas import tpu as pltpu


def _pool_kernel(feat_ref, mask_ref, out_ref):
    feat = feat_ref[...].astype(jnp.float32)
    m2 = mask_ref[...].astype(jnp.float32)
    mask = m2[:, :, None]
    num = jnp.sum(feat * mask, axis=1)
    den = jnp.sum(m2, axis=1, keepdims=True)
    out_ref[...] = (num * (1.0 / den)).astype(out_ref.dtype)


def kernel(features, input_mask):
    B, S, H = features.shape
    itemsize = jnp.dtype(features.dtype).itemsize

    tb = B
    for cand in (16, 8, 4, 2, 1):
        blk = cand * S * H * itemsize
        if B % cand == 0 and 2 * blk <= 49 << 20:
            tb = cand
            break

    grid = (B // tb,)
    feat_spec = pl.BlockSpec((tb, S, H), lambda i: (i, 0, 0))
    mask_spec = pl.BlockSpec((tb, S), lambda i: (i, 0))
    out_spec = pl.BlockSpec((tb, H), lambda i: (i, 0))

    feat_blk = tb * S * H * itemsize
    vmem = min(56 << 20, 2 * feat_blk + (8 << 20))

    return pl.pallas_call(
        _pool_kernel,
        out_shape=jax.ShapeDtypeStruct((B, H), features.dtype),
        grid=grid,
        in_specs=[feat_spec, mask_spec],
        out_specs=out_spec,
        compiler_params=pltpu.CompilerParams(
            dimension_semantics=("parallel",),
            vmem_limit_bytes=int(vmem),
        ),
        cost_estimate=pl.CostEstimate(
            flops=2 * B * S * H,
            transcendentals=0,
            bytes_accessed=B * S * H * itemsize + B * S * 4 + B * H * itemsize,
        ),
    )(features, input_mask)
